# Initial kernel scaffold; baseline (speedup 1.0000x reference)
import jax
import jax.numpy as jnp
from jax.experimental import pallas as pl


def kernel(x, z, idx_pa):
    raise NotImplementedError("write your pallas kernel here")


# speedup vs baseline: 9.0733x; 9.0733x over previous
import functools
import jax
import jax.numpy as jnp
from jax import lax
from jax.experimental import pallas as pl
from jax.experimental.pallas import tpu as pltpu
from jax.experimental.pallas import tpu_sc as plsc

CHUNK = 1024


def kernel(x, z, idx_pa):
    B, N, DX = x.shape
    DZ = z.shape[2]
    S = idx_pa.shape[1]
    idx = idx_pa.astype(jnp.int32)
    n_chunks = S // CHUNK

    mesh = plsc.VectorSubcoreMesh(core_axis_name="c", subcore_axis_name="s")

    @functools.partial(
        pl.kernel,
        mesh=mesh,
        compiler_params=pltpu.CompilerParams(use_tc_tiling_on_sc=False),
        out_type=(
            jax.ShapeDtypeStruct((B, S, DX), jnp.float32),
            jax.ShapeDtypeStruct((B, S, DZ), jnp.float32),
        ),
        scratch_types=[
            pltpu.VMEM((CHUNK,), jnp.int32),
            pltpu.VMEM((CHUNK, DX), jnp.float32),
            pltpu.VMEM((CHUNK, DZ), jnp.float32),
            pltpu.SemaphoreType.DMA,
            pltpu.SemaphoreType.DMA,
        ],
    )
    def gather_kernel(x_hbm, z_hbm, idx_hbm, ox_hbm, oz_hbm,
                      idx_v, xrows_v, zrows_v, semx, semz):
        w = lax.axis_index("s") * 2 + lax.axis_index("c")

        @pl.loop(0, n_chunks)
        def _(c):
            base = c * CHUNK
            pltpu.sync_copy(idx_hbm.at[w, pl.ds(base, CHUNK)], idx_v)
            cx = pltpu.async_copy(x_hbm.at[w].at[idx_v], xrows_v, semx)
            cz = pltpu.async_copy(z_hbm.at[w].at[idx_v], zrows_v, semz)
            cx.wait()
            cz.wait()
            pltpu.sync_copy(xrows_v, ox_hbm.at[w, pl.ds(base, CHUNK)])
            pltpu.sync_copy(zrows_v, oz_hbm.at[w, pl.ds(base, CHUNK)])

    return gather_kernel(x, z, idx)


# speedup vs baseline: 9.2114x; 1.0152x over previous
import functools
import jax
import jax.numpy as jnp
from jax import lax
from jax.experimental import pallas as pl
from jax.experimental.pallas import tpu as pltpu
from jax.experimental.pallas import tpu_sc as plsc

CHUNK = 512
NSLOTS = 2


def kernel(x, z, idx_pa):
    B, N, DX = x.shape
    DZ = z.shape[2]
    S = idx_pa.shape[1]
    idx = idx_pa.astype(jnp.int32)
    n_chunks = S // CHUNK

    mesh = plsc.VectorSubcoreMesh(core_axis_name="c", subcore_axis_name="s")

    @functools.partial(
        pl.kernel,
        mesh=mesh,
        compiler_params=pltpu.CompilerParams(use_tc_tiling_on_sc=False),
        out_type=(
            jax.ShapeDtypeStruct((B, S, DX), jnp.float32),
            jax.ShapeDtypeStruct((B, S, DZ), jnp.float32),
        ),
        scratch_types=[
            pltpu.VMEM((S,), jnp.int32),
            pltpu.VMEM((NSLOTS, CHUNK, DX), jnp.float32),
            pltpu.VMEM((NSLOTS, CHUNK, DZ), jnp.float32),
            pltpu.SemaphoreType.DMA((NSLOTS,)),
            pltpu.SemaphoreType.DMA((NSLOTS,)),
            pltpu.SemaphoreType.DMA,
        ],
    )
    def gather_kernel(x_hbm, z_hbm, idx_hbm, ox_hbm, oz_hbm,
                      idx_v, xrows_v, zrows_v, gsem, wsem, isem):
        w = lax.axis_index("s") * 2 + lax.axis_index("c")
        pltpu.async_copy(idx_hbm.at[w], idx_v, isem).wait()

        def gather_start(c, slot):
            pltpu.async_copy(
                x_hbm.at[w].at[idx_v.at[pl.ds(c * CHUNK, CHUNK)]],
                xrows_v.at[slot], gsem.at[slot])
            pltpu.async_copy(
                z_hbm.at[w].at[idx_v.at[pl.ds(c * CHUNK, CHUNK)]],
                zrows_v.at[slot], gsem.at[slot])

        def gather_wait(c, slot):
            pltpu.make_async_copy(
                x_hbm.at[w].at[idx_v.at[pl.ds(c * CHUNK, CHUNK)]],
                xrows_v.at[slot], gsem.at[slot]).wait()
            pltpu.make_async_copy(
                z_hbm.at[w].at[idx_v.at[pl.ds(c * CHUNK, CHUNK)]],
                zrows_v.at[slot], gsem.at[slot]).wait()

        def wb_start(c, slot):
            pltpu.async_copy(
                xrows_v.at[slot], ox_hbm.at[w, pl.ds(c * CHUNK, CHUNK)],
                wsem.at[slot])
            pltpu.async_copy(
                zrows_v.at[slot], oz_hbm.at[w, pl.ds(c * CHUNK, CHUNK)],
                wsem.at[slot])

        def wb_wait(c, slot):
            pltpu.make_async_copy(
                xrows_v.at[slot], ox_hbm.at[w, pl.ds(c * CHUNK, CHUNK)],
                wsem.at[slot]).wait()
            pltpu.make_async_copy(
                zrows_v.at[slot], oz_hbm.at[w, pl.ds(c * CHUNK, CHUNK)],
                wsem.at[slot]).wait()

        gather_start(0, 0)
        gather_start(1, 1)

        @pl.loop(0, n_chunks - NSLOTS, step=NSLOTS)
        def _(c):
            gather_wait(c, 0)
            wb_start(c, 0)
            gather_wait(c + 1, 1)
            wb_start(c + 1, 1)
            wb_wait(c, 0)
            gather_start(c + 2, 0)
            wb_wait(c + 1, 1)
            gather_start(c + 3, 1)

        c = n_chunks - NSLOTS
        gather_wait(c, 0)
        wb_start(c, 0)
        gather_wait(c + 1, 1)
        wb_start(c + 1, 1)
        wb_wait(c, 0)
        wb_wait(c + 1, 1)

    return gather_kernel(x, z, idx)


# speedup vs baseline: 9.2252x; 1.0015x over previous
import functools
import jax
import jax.numpy as jnp
from jax import lax
from jax.experimental import pallas as pl
from jax.experimental.pallas import tpu as pltpu
from jax.experimental.pallas import tpu_sc as plsc

CHUNK = 256
NSLOTS = 4


def kernel(x, z, idx_pa):
    B, N, DX = x.shape
    DZ = z.shape[2]
    S = idx_pa.shape[1]
    idx = idx_pa.astype(jnp.int32)
    n_chunks = S // CHUNK

    mesh = plsc.VectorSubcoreMesh(core_axis_name="c", subcore_axis_name="s")

    @functools.partial(
        pl.kernel,
        mesh=mesh,
        compiler_params=pltpu.CompilerParams(use_tc_tiling_on_sc=False),
        out_type=(
            jax.ShapeDtypeStruct((B, S, DX), jnp.float32),
            jax.ShapeDtypeStruct((B, S, DZ), jnp.float32),
        ),
        scratch_types=[
            pltpu.VMEM((S,), jnp.int32),
            pltpu.VMEM((NSLOTS, CHUNK, DX), jnp.float32),
            pltpu.VMEM((NSLOTS, CHUNK, DZ), jnp.float32),
            pltpu.SemaphoreType.DMA((NSLOTS,)),
            pltpu.SemaphoreType.DMA((NSLOTS,)),
            pltpu.SemaphoreType.DMA,
        ],
    )
    def gather_kernel(x_hbm, z_hbm, idx_hbm, ox_hbm, oz_hbm,
                      idx_v, xrows_v, zrows_v, gsem, wsem, isem):
        w = lax.axis_index("s") * 2 + lax.axis_index("c")
        pltpu.async_copy(idx_hbm.at[w], idx_v, isem).wait()

        def gather_start(c, slot):
            pltpu.async_copy(
                x_hbm.at[w].at[idx_v.at[pl.ds(c * CHUNK, CHUNK)]],
                xrows_v.at[slot], gsem.at[slot])
            pltpu.async_copy(
                z_hbm.at[w].at[idx_v.at[pl.ds(c * CHUNK, CHUNK)]],
                zrows_v.at[slot], gsem.at[slot])

        def gather_wait(c, slot):
            pltpu.make_async_copy(
                x_hbm.at[w].at[idx_v.at[pl.ds(c * CHUNK, CHUNK)]],
                xrows_v.at[slot], gsem.at[slot]).wait()
            pltpu.make_async_copy(
                z_hbm.at[w].at[idx_v.at[pl.ds(c * CHUNK, CHUNK)]],
                zrows_v.at[slot], gsem.at[slot]).wait()

        def wb_start(c, slot):
            pltpu.async_copy(
                xrows_v.at[slot], ox_hbm.at[w, pl.ds(c * CHUNK, CHUNK)],
                wsem.at[slot])
            pltpu.async_copy(
                zrows_v.at[slot], oz_hbm.at[w, pl.ds(c * CHUNK, CHUNK)],
                wsem.at[slot])

        def wb_wait(c, slot):
            pltpu.make_async_copy(
                xrows_v.at[slot], ox_hbm.at[w, pl.ds(c * CHUNK, CHUNK)],
                wsem.at[slot]).wait()
            pltpu.make_async_copy(
                zrows_v.at[slot], oz_hbm.at[w, pl.ds(c * CHUNK, CHUNK)],
                wsem.at[slot]).wait()

        for k in range(NSLOTS):
            gather_start(k, k)

        @pl.loop(0, n_chunks - NSLOTS, step=NSLOTS)
        def _(c):
            for k in range(NSLOTS):
                gather_wait(c + k, k)
                wb_start(c + k, k)
            for k in range(NSLOTS):
                wb_wait(c + k, k)
                gather_start(c + NSLOTS + k, k)

        c = n_chunks - NSLOTS
        for k in range(NSLOTS):
            gather_wait(c + k, k)
            wb_start(c + k, k)
        for k in range(NSLOTS):
            wb_wait(c + k, k)

    return gather_kernel(x, z, idx)
